# baseline (speedup 1.0000x reference)
import functools

import jax
import jax.numpy as jnp
import numpy as np
from jax import lax
from jax.experimental import pallas as pl
from jax.experimental.pallas import tpu as pltpu
from jax.experimental.pallas import tpu_sc as plsc

_B = 4096
_D = 64
_K = 5
_NC = 2
_NS = 16
_NW = _NC * _NS
_CHUNK = 128
_BLK = 8192
_Q = 4
_HIMASK = np.uint32(0xFFFF0000)


def _pack_trunc(hi_f32, lo_f32):
    ra = lax.bitcast_convert_type(hi_f32, jnp.uint32)
    rb = lax.bitcast_convert_type(lo_f32, jnp.uint32)
    return lax.bitcast_convert_type((ra & _HIMASK) | (rb >> 16), jnp.float32)


def _tc_repack(table_t):
    v = table_t.shape[1]
    nblk = (v + _BLK - 1) // _BLK
    h_blk = (v // _Q + _BLK - 1) // _BLK
    h = h_blk * _BLK

    def body(q0_ref, q1_ref, q2_ref, q3_ref, o_ref):
        c01 = _pack_trunc(q0_ref[...], q1_ref[...]).T
        c23 = _pack_trunc(q2_ref[...], q3_ref[...]).T
        o_ref[...] = jnp.concatenate([c01, c23], axis=1)

    def mk_map(q):
        return lambda i: (0, jnp.minimum(i + q * h_blk, nblk - 1))

    return pl.pallas_call(
        body,
        grid=(h_blk,),
        in_specs=[pl.BlockSpec((_D, _BLK), mk_map(q)) for q in range(_Q)],
        out_specs=pl.BlockSpec((_BLK, 2 * _D), lambda i: (i, 0)),
        out_shape=jax.ShapeDtypeStruct((h, 2 * _D), jnp.float32),
    )(table_t, table_t, table_t, table_t)


def _sc_gather_one(table, idx):
    n = idx.shape[0]
    per_w = n // _NW
    w = table.shape[1]
    mesh = plsc.VectorSubcoreMesh(core_axis_name="c", subcore_axis_name="s")

    @functools.partial(
        pl.kernel,
        mesh=mesh,
        out_type=jax.ShapeDtypeStruct((n, w), jnp.float32),
        scratch_types=[
            pltpu.VMEM((per_w,), jnp.int32),
            pltpu.VMEM((per_w, w), jnp.float32),
            pltpu.SemaphoreType.DMA,
        ],
    )
    def gather_kernel(tab_hbm, i_hbm, rows_out, i_v, rows_v, sem):
        wid = lax.axis_index("s") * _NC + lax.axis_index("c")
        base = wid * per_w
        pltpu.sync_copy(i_hbm.at[pl.ds(base, per_w)], i_v)
        copies = []
        for j in range(per_w // _CHUNK):
            copies.append(pltpu.async_copy(
                tab_hbm.at[i_v.at[pl.ds(j * _CHUNK, _CHUNK)]],
                rows_v.at[pl.ds(j * _CHUNK, _CHUNK)], sem))
        for c in copies:
            c.wait()
        pltpu.sync_copy(rows_v, rows_out.at[pl.ds(base, per_w)])

    return gather_kernel(table, idx)


def _extract(rows_ref, qh, qa, lo, hi):
    w = jnp.where(qh > 0.5, rows_ref[lo:hi, _D:], rows_ref[lo:hi, :_D])
    wu = lax.bitcast_convert_type(w, jnp.uint32)
    return lax.bitcast_convert_type(
        jnp.where(qa > 0.5, wu << 16, wu & _HIMASK), jnp.float32)


def _tc_loss(t_rows, cn_rows, qt, qcn):
    def body(t_ref, cn_ref, qt_ref, qcn_ref, o_ref):
        t = _extract(t_ref, qt_ref[:, 0:1], qt_ref[:, 1:2], 0, _B)
        c = _extract(cn_ref, qcn_ref[0:_B, 0:1], qcn_ref[0:_B, 1:2], 0, _B)
        acc = jnp.log(jax.nn.sigmoid(jnp.sum(t * c, axis=1)))
        for k in range(_K):
            lo, hi = _B * (k + 1), _B * (k + 2)
            n = _extract(cn_ref, qcn_ref[lo:hi, 0:1], qcn_ref[lo:hi, 1:2],
                         lo, hi)
            acc = acc + jnp.log(jax.nn.sigmoid(-jnp.sum(t * n, axis=1)))
        o_ref[0, 0] = -jnp.sum(acc) / _B

    out = pl.pallas_call(
        body,
        out_shape=jax.ShapeDtypeStruct((1, 1), jnp.float32),
        out_specs=pl.BlockSpec(memory_space=pltpu.SMEM),
    )(t_rows, cn_rows, qt, qcn)
    return out[0, 0]


def kernel(target, context, neg_samples, in_embed, out_embed):
    v = in_embed.shape[0]
    h = ((v // _Q + _BLK - 1) // _BLK) * _BLK
    in2 = _tc_repack(in_embed.T)
    out2 = _tc_repack(out_embed.T)
    idx_t = target.astype(jnp.int32)
    idx_cn = jnp.concatenate(
        [context.astype(jnp.int32), neg_samples.astype(jnp.int32).T.reshape(-1)])

    def split(idx):
        q = idx // h
        j = idx - q * h
        flags = jnp.stack([(q >= 2).astype(jnp.float32),
                           (q & 1).astype(jnp.float32)], axis=1)
        return j, flags

    j_t, qt = split(idx_t)
    j_cn, qcn = split(idx_cn)
    t_rows = _sc_gather_one(in2, j_t)
    cn_rows = _sc_gather_one(out2, j_cn)
    return _tc_loss(t_rows, cn_rows, qt, qcn)


# speedup vs baseline: 1.4216x; 1.4216x over previous
import functools

import jax
import jax.numpy as jnp
import numpy as np
from jax import lax
from jax.experimental import pallas as pl
from jax.experimental.pallas import tpu as pltpu
from jax.experimental.pallas import tpu_sc as plsc

_B = 4096
_D = 64
_K = 5
_NC = 2
_NS = 16
_NW = _NC * _NS
_CHUNK = 128
_BLK = 8192
_Q = 4
_HIMASK = np.uint32(0xFFFF0000)


def _pack_trunc(hi_f32, lo_f32):
    ra = lax.bitcast_convert_type(hi_f32, jnp.uint32)
    rb = lax.bitcast_convert_type(lo_f32, jnp.uint32)
    return lax.bitcast_convert_type((ra & _HIMASK) | (rb >> 16), jnp.float32)


def _tc_repack(table_t):
    v = table_t.shape[1]
    nblk = (v + _BLK - 1) // _BLK
    h_blk = (v // _Q + _BLK - 1) // _BLK
    h = h_blk * _BLK

    def body(q0_ref, q1_ref, q2_ref, q3_ref, o_ref):
        c01 = _pack_trunc(q0_ref[...], q1_ref[...]).T
        c23 = _pack_trunc(q2_ref[...], q3_ref[...]).T
        o_ref[...] = jnp.concatenate([c01, c23], axis=1)

    def mk_map(q):
        return lambda i: (0, jnp.minimum(i + q * h_blk, nblk - 1))

    return pl.pallas_call(
        body,
        grid=(h_blk,),
        in_specs=[pl.BlockSpec((_D, _BLK), mk_map(q)) for q in range(_Q)],
        out_specs=pl.BlockSpec((_BLK, 2 * _D), lambda i: (i, 0)),
        out_shape=jax.ShapeDtypeStruct((h, 2 * _D), jnp.float32),
    )(table_t, table_t, table_t, table_t)


def _sc_gather_one(table, idx):
    n = idx.shape[0]
    per_w = n // _NW
    w = table.shape[1]
    mesh = plsc.VectorSubcoreMesh(core_axis_name="c", subcore_axis_name="s")

    @functools.partial(
        pl.kernel,
        mesh=mesh,
        out_type=jax.ShapeDtypeStruct((n, w), jnp.float32),
        scratch_types=[
            pltpu.VMEM((per_w,), jnp.int32),
            pltpu.VMEM((per_w, w), jnp.float32),
            pltpu.SemaphoreType.DMA,
        ],
    )
    def gather_kernel(tab_hbm, i_hbm, rows_out, i_v, rows_v, sem):
        wid = lax.axis_index("s") * _NC + lax.axis_index("c")
        base = wid * per_w
        pltpu.sync_copy(i_hbm.at[pl.ds(base, per_w)], i_v)
        copies = []
        for j in range(per_w // _CHUNK):
            copies.append(pltpu.async_copy(
                tab_hbm.at[i_v.at[pl.ds(j * _CHUNK, _CHUNK)]],
                rows_v.at[pl.ds(j * _CHUNK, _CHUNK)], sem))
        for c in copies:
            c.wait()
        pltpu.sync_copy(rows_v, rows_out.at[pl.ds(base, per_w)])

    return gather_kernel(table, idx)


_NBUF = 8


def _sc_tile_gather(table_t, idx):
    n = idx.shape[0]
    per_w = n // _NW
    d = table_t.shape[0]
    mesh = plsc.VectorSubcoreMesh(core_axis_name="c", subcore_axis_name="s")

    @functools.partial(
        pl.kernel,
        mesh=mesh,
        compiler_params=pltpu.CompilerParams(needs_layout_passes=False),
        out_type=jax.ShapeDtypeStruct((n, d), jnp.float32),
        scratch_types=(
            [pltpu.VMEM((per_w + 16,), jnp.int32),
             pltpu.VMEM((per_w, d), jnp.float32)]
            + [pltpu.VMEM((d, 128), jnp.float32) for _ in range(_NBUF)]
            + [pltpu.SemaphoreType.DMA for _ in range(_NBUF)]
        ),
    )
    def k(tab_hbm, idx_hbm, out_hbm, idx_v, rows_v, *bufsem):
        bufs = bufsem[:_NBUF]
        sems = bufsem[_NBUF:]
        wid = lax.axis_index("s") * _NC + lax.axis_index("c")
        base = wid * per_w
        pltpu.sync_copy(idx_hbm.at[pl.ds(base, per_w)], idx_v.at[pl.ds(0, per_w)])

        def win(i):
            v = idx_v[pl.ds(i, 16)][0]
            return v, pl.multiple_of((v >> 7) << 7, 128)

        def start(i, b):
            _, c = win(i)
            return pltpu.async_copy(tab_hbm.at[:, pl.ds(c, 128)], bufs[b], sems[b])

        for b in range(_NBUF):
            start(b, b)

        @pl.loop(0, per_w, step=_NBUF)
        def _(g):
            for b in range(_NBUF):
                i = g + b
                v, _c = win(i)
                l = v & 127
                pltpu.make_async_copy(tab_hbm.at[:, pl.ds(_c, 128)],
                                      bufs[b], sems[b]).wait()
                for t16 in range(d // 16):
                    rows = lax.iota(jnp.int32, 16) + (16 * t16)
                    cols = jnp.full((16,), l, jnp.int32)
                    rows_v[i, pl.ds(16 * t16, 16)] = plsc.load_gather(
                        bufs[b], [rows, cols])

                @pl.when(g + _NBUF + b < per_w)
                def _():
                    start(g + _NBUF + b, b)

        pltpu.sync_copy(rows_v, out_hbm.at[pl.ds(base, per_w)])

    return k(table_t, idx)


def _extract(rows_ref, qh, qa, lo, hi):
    w = jnp.where(qh > 0.5, rows_ref[lo:hi, _D:], rows_ref[lo:hi, :_D])
    wu = lax.bitcast_convert_type(w, jnp.uint32)
    return lax.bitcast_convert_type(
        jnp.where(qa > 0.5, wu << 16, wu & _HIMASK), jnp.float32)


def _tc_loss(t_rows, cn_rows, qcn):
    def body(t_ref, cn_ref, qcn_ref, o_ref):
        t = t_ref[...]
        c = _extract(cn_ref, qcn_ref[0:_B, 0:1], qcn_ref[0:_B, 1:2], 0, _B)
        acc = jnp.log(jax.nn.sigmoid(jnp.sum(t * c, axis=1)))
        for k in range(_K):
            lo, hi = _B * (k + 1), _B * (k + 2)
            n = _extract(cn_ref, qcn_ref[lo:hi, 0:1], qcn_ref[lo:hi, 1:2],
                         lo, hi)
            acc = acc + jnp.log(jax.nn.sigmoid(-jnp.sum(t * n, axis=1)))
        o_ref[0, 0] = -jnp.sum(acc) / _B

    out = pl.pallas_call(
        body,
        out_shape=jax.ShapeDtypeStruct((1, 1), jnp.float32),
        out_specs=pl.BlockSpec(memory_space=pltpu.SMEM),
    )(t_rows, cn_rows, qcn)
    return out[0, 0]


def kernel(target, context, neg_samples, in_embed, out_embed):
    v = in_embed.shape[0]
    h = ((v // _Q + _BLK - 1) // _BLK) * _BLK
    out2 = _tc_repack(out_embed.T)
    idx_t = target.astype(jnp.int32)
    idx_cn = jnp.concatenate(
        [context.astype(jnp.int32), neg_samples.astype(jnp.int32).T.reshape(-1)])

    def split(idx):
        q = idx // h
        j = idx - q * h
        flags = jnp.stack([(q >= 2).astype(jnp.float32),
                           (q & 1).astype(jnp.float32)], axis=1)
        return j, flags

    j_cn, qcn = split(idx_cn)
    t_rows = _sc_tile_gather(in_embed.T, idx_t)
    cn_rows = _sc_gather_one(out2, j_cn)
    return _tc_loss(t_rows, cn_rows, qcn)
